# baseline (speedup 1.0000x reference)
import jax
import jax.numpy as jnp
from jax import lax
from jax.experimental import pallas as pl
from jax.experimental.pallas import tpu as pltpu
from jax.experimental.pallas import tpu_sc as plsc

N = 10000
D = 128
E = 320000

NC = 2
NS = 16
NW = NC * NS
CHUNK = 128
EPT = 10240
HALVES = 2
HEDGES = EPT // HALVES
HROWS = HEDGES // CHUNK
CPH = HEDGES // CHUNK
EPAD = NW * EPT
NACC = 10240
ROWS_PER_TILE = NACC // NS

_mesh = plsc.VectorSubcoreMesh(core_axis_name="c", subcore_axis_name="s")


def _spmm_body(x_hbm, rows_hbm, cols_hbm, vals_hbm, out_hbm,
               cols_v, rows_v, vals_v, buf0, buf1, acc,
               semg0, semg1, sems0):
    c = lax.axis_index("c")
    s = lax.axis_index("s")
    wid = s * NC + c

    with jax.named_scope("acc_zero"):
        def zero_body(e, carry):
            for q in range(D // 16):
                buf0[e, pl.ds(q * 16, 16)] = jnp.zeros((16,), jnp.float32)
            return carry
        lax.fori_loop(0, CHUNK, zero_body, 0)
        r0 = s * ROWS_PER_TILE
        for z in range(ROWS_PER_TILE // CHUNK):
            pltpu.sync_copy(buf0.at[pl.ds(0, CHUNK)],
                            acc.at[pl.ds(r0 + z * CHUNK, CHUNK)])
        plsc.subcore_barrier()

    nrows_idx = EPT // CHUNK

    def scale(buf, ebase):
        def scale_body(g, inner):
            vv = vals_v[pl.ds(ebase + g * 16, 16)]
            for i in range(16):
                v = vv[i]
                e = g * 16 + i
                for q in range(D // 16):
                    sl = pl.ds(q * 16, 16)
                    buf[e, sl] = buf[e, sl] * v
            return inner
        lax.fori_loop(0, CHUNK // 16, scale_body, 0)

    with jax.named_scope("edge_loop"):
        for h in range(HALVES):
            rbase = wid * nrows_idx + h * HROWS
            pltpu.sync_copy(cols_hbm.at[pl.ds(rbase, HROWS)], cols_v)
            pltpu.sync_copy(rows_hbm.at[pl.ds(rbase, HROWS)], rows_v)
            pltpu.sync_copy(
                vals_hbm.at[pl.ds(wid * EPT + h * HEDGES, HEDGES)], vals_v)

            pltpu.async_copy(x_hbm.at[cols_v.at[0]], buf0, semg0)

            def pair_body(tt, carry):
                a0 = 2 * tt
                pltpu.make_async_copy(x_hbm.at[cols_v.at[a0]],
                                      buf0, semg0).wait()
                pltpu.async_copy(x_hbm.at[cols_v.at[a0 + 1]], buf1, semg1)
                scale(buf0, a0 * CHUNK)
                s0 = pltpu.async_copy(buf0, acc.at[rows_v.at[a0]], sems0,
                                      add=True)
                pltpu.make_async_copy(x_hbm.at[cols_v.at[a0 + 1]],
                                      buf1, semg1).wait()
                scale(buf1, (a0 + 1) * CHUNK)
                s0.wait()

                @pl.when(tt < CPH // 2 - 1)
                def _():
                    pltpu.async_copy(x_hbm.at[cols_v.at[a0 + 2]], buf0, semg0)
                pltpu.sync_copy(buf1, acc.at[rows_v.at[a0 + 1]], add=True)
                return carry
            lax.fori_loop(0, CPH // 2, pair_body, 0)

    with jax.named_scope("writeout"):
        plsc.subcore_barrier()
        pltpu.sync_copy(acc.at[pl.ds(r0, ROWS_PER_TILE)],
                        out_hbm.at[c].at[pl.ds(r0, ROWS_PER_TILE)])


_spmm_call = pl.kernel(
    _spmm_body,
    jax.ShapeDtypeStruct((NC, NACC, D), jnp.float32),
    mesh=_mesh,
    scratch_types=[
        pltpu.VMEM((HROWS, CHUNK), jnp.int32),
        pltpu.VMEM((HROWS, CHUNK), jnp.int32),
        pltpu.VMEM((HEDGES,), jnp.float32),
        pltpu.VMEM((CHUNK, D), jnp.float32),
        pltpu.VMEM((CHUNK, D), jnp.float32),
        pltpu.VMEM_SHARED((NACC, D), jnp.float32),
        pltpu.SemaphoreType.DMA,
        pltpu.SemaphoreType.DMA,
        pltpu.SemaphoreType.DMA,
    ],
)


BN = 1000


def _dense_a_body(x_ref, w1t_ref, bias_ref, out_ref):
    out_ref[...] = (jnp.dot(x_ref[...], w1t_ref[...],
                            preferred_element_type=jnp.float32)
                    + bias_ref[...])


def _dense_a(x, w1t, bias):
    return pl.pallas_call(
        _dense_a_body,
        grid=(N // BN,),
        in_specs=[
            pl.BlockSpec((BN, D), lambda i: (i, 0)),
            pl.BlockSpec((D, D), lambda i: (0, 0)),
            pl.BlockSpec((1, D), lambda i: (0, 0)),
        ],
        out_specs=pl.BlockSpec((BN, D), lambda i: (i, 0)),
        out_shape=jax.ShapeDtypeStruct((N, D), jnp.float32),
    )(x, w1t, bias.reshape(1, D))


def _dense_b_body(ya_ref, part_ref, w2t_ref, a_ref, out_ref):
    p = part_ref[0] + part_ref[1]
    y = ya_ref[...] + jnp.dot(p, w2t_ref[...],
                              preferred_element_type=jnp.float32)
    a = a_ref[0]
    out_ref[...] = jnp.where(y >= 0.0, y, a * y)


def _dense_b(ya, part, w2t, a):
    return pl.pallas_call(
        _dense_b_body,
        grid=(N // BN,),
        in_specs=[
            pl.BlockSpec((BN, D), lambda i: (i, 0)),
            pl.BlockSpec((NC, BN, D), lambda i: (0, i, 0)),
            pl.BlockSpec((D, D), lambda i: (0, 0)),
            pl.BlockSpec(memory_space=pltpu.SMEM),
        ],
        out_specs=pl.BlockSpec((BN, D), lambda i: (i, 0)),
        out_shape=jax.ShapeDtypeStruct((N, D), jnp.float32),
    )(ya, part, w2t, a.reshape(1))


def kernel(X, edge_index, edge_vals, W1_0, b1_0, W2_0, b2_0, a_0,
           W1_1, b1_1, W2_1, b2_1, a_1):
    pad = EPAD - E
    pad_rows = N + (jnp.arange(pad, dtype=jnp.int32) % (NACC - N))
    pad_cols = jnp.arange(pad, dtype=jnp.int32) % N
    rows2 = jnp.concatenate(
        [edge_index[0], pad_rows]).reshape(EPAD // CHUNK, CHUNK)
    cols2 = jnp.concatenate(
        [edge_index[1], pad_cols]).reshape(EPAD // CHUNK, CHUNK)
    vals1 = jnp.concatenate([edge_vals, jnp.zeros((pad,), jnp.float32)])

    part = _spmm_call(X, rows2, cols2, vals1)
    ya = _dense_a(X, W1_0.T, b1_0 + b2_0)
    t1 = _dense_b(ya, part, W2_0.T, a_0)
    part = _spmm_call(t1, rows2, cols2, vals1)
    ya = _dense_a(t1, W1_1.T, b1_1 + b2_1)
    t2 = _dense_b(ya, part, W2_1.T, a_1)
    return jnp.expand_dims(t2, 0)


# speedup vs baseline: 1.0931x; 1.0931x over previous
import jax
import jax.numpy as jnp
from jax import lax
from jax.experimental import pallas as pl
from jax.experimental.pallas import tpu as pltpu
from jax.experimental.pallas import tpu_sc as plsc

N = 10000
D = 128
E = 320000

NC = 2
NS = 16
NW = NC * NS
CHUNK = 128
EPT = 10240
QUARTERS = 5
QEDGES = EPT // QUARTERS
QROWS = QEDGES // CHUNK
CPQ = QEDGES // CHUNK
EPAD = NW * EPT
NACC = 10240
ROWS_PER_TILE = NACC // NS

_mesh = plsc.VectorSubcoreMesh(core_axis_name="c", subcore_axis_name="s")


def _spmm_body(x_hbm, rows_hbm, cols_hbm, vals_hbm, out_hbm,
               cols_v, rows_v, vals_v, buf0, buf1, acc,
               semg0, semg1, semst):
    c = lax.axis_index("c")
    s = lax.axis_index("s")
    wid = s * NC + c
    nrows_idx = EPT // CHUNK

    def stage_copies(q, qb):
        rbase = wid * nrows_idx + q * QROWS
        return [
            (cols_hbm.at[pl.ds(rbase, QROWS)], cols_v.at[qb]),
            (rows_hbm.at[pl.ds(rbase, QROWS)], rows_v.at[qb]),
            (vals_hbm.at[pl.ds(wid * EPT + q * QEDGES, QEDGES)],
             vals_v.at[qb]),
        ]

    def stage_start(q, qb):
        for src, dst in stage_copies(q, qb):
            pltpu.async_copy(src, dst, semst)

    def stage_wait(q, qb):
        for src, dst in stage_copies(q, qb):
            pltpu.make_async_copy(src, dst, semst).wait()

    stage_start(0, 0)

    with jax.named_scope("acc_zero"):
        def zero_body(e, carry):
            for q in range(D // 16):
                buf0[e, pl.ds(q * 16, 16)] = jnp.zeros((16,), jnp.float32)
            return carry
        lax.fori_loop(0, CHUNK, zero_body, 0)
        r0 = s * ROWS_PER_TILE
        for z in range(ROWS_PER_TILE // CHUNK):
            pltpu.sync_copy(buf0.at[pl.ds(0, CHUNK)],
                            acc.at[pl.ds(r0 + z * CHUNK, CHUNK)])
        plsc.subcore_barrier()

    def scale(buf, qb, ebase):
        def scale_body(g, inner):
            vv = vals_v[qb, pl.ds(ebase + g * 16, 16)]
            for i in range(16):
                v = vv[i]
                e = g * 16 + i
                for q in range(D // 16):
                    sl = pl.ds(q * 16, 16)
                    buf[e, sl] = buf[e, sl] * v
            return inner
        lax.fori_loop(0, CHUNK // 16, scale_body, 0)

    with jax.named_scope("edge_loop"):
        for q in range(QUARTERS):
            qb = q % 2
            stage_wait(q, qb)
            cv = cols_v.at[qb]
            rv = rows_v.at[qb]
            if q == 0:
                pltpu.async_copy(x_hbm.at[cv.at[0]], buf0, semg0)
            if q + 1 < QUARTERS:
                stage_start(q + 1, 1 - qb)

            def pair_body(tt, carry, cv=cv, rv=rv, qb=qb):
                a0 = 2 * tt
                pltpu.make_async_copy(x_hbm.at[cv.at[a0]],
                                      buf0, semg0).wait()
                pltpu.async_copy(x_hbm.at[cv.at[a0 + 1]], buf1, semg1)
                scale(buf0, qb, a0 * CHUNK)
                pltpu.sync_copy(buf0, acc.at[rv.at[a0]], add=True)
                pltpu.make_async_copy(x_hbm.at[cv.at[a0 + 1]],
                                      buf1, semg1).wait()

                @pl.when(tt < CPQ // 2 - 1)
                def _():
                    pltpu.async_copy(x_hbm.at[cv.at[a0 + 2]], buf0, semg0)
                scale(buf1, qb, (a0 + 1) * CHUNK)
                pltpu.sync_copy(buf1, acc.at[rv.at[a0 + 1]], add=True)
                return carry
            lax.fori_loop(0, CPQ // 2, pair_body, 0)

            if q + 1 < QUARTERS:
                pltpu.async_copy(x_hbm.at[cols_v.at[1 - qb].at[0]],
                                 buf0, semg0)

    with jax.named_scope("writeout"):
        plsc.subcore_barrier()
        pltpu.sync_copy(acc.at[pl.ds(r0, ROWS_PER_TILE)],
                        out_hbm.at[c].at[pl.ds(r0, ROWS_PER_TILE)])


_spmm_call = pl.kernel(
    _spmm_body,
    jax.ShapeDtypeStruct((NC, NACC, D), jnp.float32),
    mesh=_mesh,
    scratch_types=[
        pltpu.VMEM((2, QROWS, CHUNK), jnp.int32),
        pltpu.VMEM((2, QROWS, CHUNK), jnp.int32),
        pltpu.VMEM((2, QEDGES), jnp.float32),
        pltpu.VMEM((CHUNK, D), jnp.float32),
        pltpu.VMEM((CHUNK, D), jnp.float32),
        pltpu.VMEM_SHARED((NACC, D), jnp.float32),
        pltpu.SemaphoreType.DMA,
        pltpu.SemaphoreType.DMA,
        pltpu.SemaphoreType.DMA,
    ],
)


BN = 1000


def _dense_body(x_ref, part_ref, w1t_ref, w2t_ref, bias_ref, a_ref, out_ref):
    p = part_ref[0] + part_ref[1]
    y = jnp.dot(x_ref[...], w1t_ref[...], preferred_element_type=jnp.float32)
    y = y + jnp.dot(p, w2t_ref[...], preferred_element_type=jnp.float32)
    y = y + bias_ref[...]
    a = a_ref[0]
    out_ref[...] = jnp.where(y >= 0.0, y, a * y)


def _dense(x, part, w1t, w2t, bias, a):
    return pl.pallas_call(
        _dense_body,
        grid=(N // BN,),
        in_specs=[
            pl.BlockSpec((BN, D), lambda i: (i, 0)),
            pl.BlockSpec((NC, BN, D), lambda i: (0, i, 0)),
            pl.BlockSpec((D, D), lambda i: (0, 0)),
            pl.BlockSpec((D, D), lambda i: (0, 0)),
            pl.BlockSpec((1, D), lambda i: (0, 0)),
            pl.BlockSpec(memory_space=pltpu.SMEM),
        ],
        out_specs=pl.BlockSpec((BN, D), lambda i: (i, 0)),
        out_shape=jax.ShapeDtypeStruct((N, D), jnp.float32),
    )(x, part, w1t, w2t, bias.reshape(1, D), a.reshape(1))


def kernel(X, edge_index, edge_vals, W1_0, b1_0, W2_0, b2_0, a_0,
           W1_1, b1_1, W2_1, b2_1, a_1):
    pad = EPAD - E
    pad_rows = N + (jnp.arange(pad, dtype=jnp.int32) % (NACC - N))
    pad_cols = jnp.arange(pad, dtype=jnp.int32) % N
    rows2 = jnp.concatenate(
        [edge_index[0], pad_rows]).reshape(EPAD // CHUNK, CHUNK)
    cols2 = jnp.concatenate(
        [edge_index[1], pad_cols]).reshape(EPAD // CHUNK, CHUNK)
    vals1 = jnp.concatenate([edge_vals, jnp.zeros((pad,), jnp.float32)])

    part = _spmm_call(X, rows2, cols2, vals1)
    t1 = _dense(X, part, W1_0.T, W2_0.T, b1_0 + b2_0, a_0)
    part = _spmm_call(t1, rows2, cols2, vals1)
    t2 = _dense(t1, part, W1_1.T, W2_1.T, b1_1 + b2_1, a_1)
    return jnp.expand_dims(t2, 0)


# speedup vs baseline: 1.1356x; 1.0388x over previous
import jax
import jax.numpy as jnp
from jax import lax
from jax.experimental import pallas as pl
from jax.experimental.pallas import tpu as pltpu
from jax.experimental.pallas import tpu_sc as plsc

N = 10000
D = 128
E = 320000

NC = 2
NS = 16
NW = NC * NS
CHUNK = 128
EPT = 10240
HALVES = 2
HEDGES = EPT // HALVES
HROWS = HEDGES // CHUNK
CPH = HEDGES // CHUNK
EPAD = NW * EPT
NACC = 10240
ROWS_PER_TILE = NACC // NS

_mesh = plsc.VectorSubcoreMesh(core_axis_name="c", subcore_axis_name="s")


def _spmm_body(x_hbm, rows_hbm, cols_hbm, vals_hbm, out_hbm,
               cols_v, rows_v, vals_v, buf0, buf1, acc, semg0, semg1):
    c = lax.axis_index("c")
    s = lax.axis_index("s")
    wid = s * NC + c

    with jax.named_scope("acc_zero"):
        def zero_body(e, carry):
            for q in range(D // 16):
                buf0[e, pl.ds(q * 16, 16)] = jnp.zeros((16,), jnp.float32)
            return carry
        lax.fori_loop(0, CHUNK, zero_body, 0)
        r0 = s * ROWS_PER_TILE
        for z in range(ROWS_PER_TILE // CHUNK):
            pltpu.sync_copy(buf0.at[pl.ds(0, CHUNK)],
                            acc.at[pl.ds(r0 + z * CHUNK, CHUNK)])
        plsc.subcore_barrier()

    nrows_idx = EPT // CHUNK

    def scale(buf, ebase):
        def scale_body(g, inner):
            vv = vals_v[pl.ds(ebase + g * 16, 16)]
            for i in range(16):
                v = vv[i]
                e = g * 16 + i
                for q in range(D // 16):
                    sl = pl.ds(q * 16, 16)
                    buf[e, sl] = buf[e, sl] * v
            return inner
        lax.fori_loop(0, CHUNK // 16, scale_body, 0)

    with jax.named_scope("edge_loop"):
        for h in range(HALVES):
            rbase = wid * nrows_idx + h * HROWS
            pltpu.sync_copy(cols_hbm.at[pl.ds(rbase, HROWS)], cols_v)
            pltpu.sync_copy(rows_hbm.at[pl.ds(rbase, HROWS)], rows_v)
            pltpu.sync_copy(
                vals_hbm.at[pl.ds(wid * EPT + h * HEDGES, HEDGES)], vals_v)

            pltpu.async_copy(x_hbm.at[cols_v.at[0]], buf0, semg0)

            def pair_body(tt, carry):
                a0 = 2 * tt
                pltpu.make_async_copy(x_hbm.at[cols_v.at[a0]],
                                      buf0, semg0).wait()
                pltpu.async_copy(x_hbm.at[cols_v.at[a0 + 1]], buf1, semg1)
                scale(buf0, a0 * CHUNK)
                pltpu.sync_copy(buf0, acc.at[rows_v.at[a0]], add=True)
                pltpu.make_async_copy(x_hbm.at[cols_v.at[a0 + 1]],
                                      buf1, semg1).wait()

                @pl.when(tt < CPH // 2 - 1)
                def _():
                    pltpu.async_copy(x_hbm.at[cols_v.at[a0 + 2]], buf0, semg0)
                scale(buf1, (a0 + 1) * CHUNK)
                pltpu.sync_copy(buf1, acc.at[rows_v.at[a0 + 1]], add=True)
                return carry
            lax.fori_loop(0, CPH // 2, pair_body, 0)

    with jax.named_scope("writeout"):
        plsc.subcore_barrier()
        pltpu.sync_copy(acc.at[pl.ds(r0, ROWS_PER_TILE)],
                        out_hbm.at[c].at[pl.ds(r0, ROWS_PER_TILE)])


_spmm_call = pl.kernel(
    _spmm_body,
    jax.ShapeDtypeStruct((NC, NACC, D), jnp.float32),
    mesh=_mesh,
    scratch_types=[
        pltpu.VMEM((HROWS, CHUNK), jnp.int32),
        pltpu.VMEM((HROWS, CHUNK), jnp.int32),
        pltpu.VMEM((HEDGES,), jnp.float32),
        pltpu.VMEM((CHUNK, D), jnp.float32),
        pltpu.VMEM((CHUNK, D), jnp.float32),
        pltpu.VMEM_SHARED((NACC, D), jnp.float32),
        pltpu.SemaphoreType.DMA,
        pltpu.SemaphoreType.DMA,
    ],
)


BN = 2000


def _dense_body(x_ref, part_ref, w1t_ref, w2t_ref, bias_ref, a_ref, out_ref):
    p = part_ref[0] + part_ref[1]
    y = jnp.dot(x_ref[...], w1t_ref[...], preferred_element_type=jnp.float32)
    y = y + jnp.dot(p, w2t_ref[...], preferred_element_type=jnp.float32)
    y = y + bias_ref[...]
    a = a_ref[0]
    out_ref[...] = jnp.where(y >= 0.0, y, a * y)


def _dense(x, part, w1t, w2t, bias, a):
    return pl.pallas_call(
        _dense_body,
        grid=(N // BN,),
        in_specs=[
            pl.BlockSpec((BN, D), lambda i: (i, 0)),
            pl.BlockSpec((NC, BN, D), lambda i: (0, i, 0)),
            pl.BlockSpec((D, D), lambda i: (0, 0)),
            pl.BlockSpec((D, D), lambda i: (0, 0)),
            pl.BlockSpec((1, D), lambda i: (0, 0)),
            pl.BlockSpec(memory_space=pltpu.SMEM),
        ],
        out_specs=pl.BlockSpec((BN, D), lambda i: (i, 0)),
        out_shape=jax.ShapeDtypeStruct((N, D), jnp.float32),
    )(x, part, w1t, w2t, bias.reshape(1, D), a.reshape(1))


def kernel(X, edge_index, edge_vals, W1_0, b1_0, W2_0, b2_0, a_0,
           W1_1, b1_1, W2_1, b2_1, a_1):
    pad = EPAD - E
    pad_rows = N + (jnp.arange(pad, dtype=jnp.int32) % (NACC - N))
    pad_cols = jnp.arange(pad, dtype=jnp.int32) % N
    rows2 = jnp.concatenate(
        [edge_index[0], pad_rows]).reshape(EPAD // CHUNK, CHUNK)
    cols2 = jnp.concatenate(
        [edge_index[1], pad_cols]).reshape(EPAD // CHUNK, CHUNK)
    vals1 = jnp.concatenate([edge_vals, jnp.zeros((pad,), jnp.float32)])

    part = _spmm_call(X, rows2, cols2, vals1)
    t1 = _dense(X, part, W1_0.T, W2_0.T, b1_0 + b2_0, a_0)
    part = _spmm_call(t1, rows2, cols2, vals1)
    t2 = _dense(t1, part, W1_1.T, W2_1.T, b1_1 + b2_1, a_1)
    return jnp.expand_dims(t2, 0)
